# baseline (speedup 1.0000x reference)
import functools
import math

import jax
import jax.numpy as jnp
from jax import lax
from jax.experimental import pallas as pl
from jax.experimental.pallas import tpu as pltpu
from jax.experimental.pallas import tpu_sc as plsc

EPS = 1e-5
_NC = 2
_NS = 16
_LANES = 16


def _pick_chunk(E):
    for ch in range(2048, 15, -16):
        if E % ch == 0 and (E // ch) % 2 == 0:
            return ch
    raise ValueError(f"no valid chunk for E={E}")


def _sc_edge_pass(D, E, N, inv_scale, write_m):
    CH = _pick_chunk(E)
    NCH = E // CH
    NV = CH // _LANES
    n_tiles_active = (D + 1) // 2

    mesh = plsc.VectorSubcoreMesh(
        core_axis_name="c", subcore_axis_name="s",
        num_cores=_NC, num_subcores=_NS)

    out_type = [jax.ShapeDtypeStruct((D * N,), jnp.float32)]
    if write_m:
        out_type = [
            jax.ShapeDtypeStruct((D * E,), jnp.float32),
            jax.ShapeDtypeStruct((D * N,), jnp.float32),
            jax.ShapeDtypeStruct((D * 16,), jnp.float32),
            jax.ShapeDtypeStruct((D * 16,), jnp.float32),
        ]

    scratch = dict(
        ta0=pltpu.VMEM((N,), jnp.float32),
        ta1=pltpu.VMEM((N,), jnp.float32),
        tb0=pltpu.VMEM((N,), jnp.float32),
        tb1=pltpu.VMEM((N,), jnp.float32),
        td0=pltpu.VMEM((N,), jnp.float32),
        td1=pltpu.VMEM((N,), jnp.float32),
        agg0=pltpu.VMEM((N,), jnp.float32),
        agg1=pltpu.VMEM((N,), jnp.float32),
        idx0=pltpu.VMEM((CH,), jnp.int32),
        idx1=pltpu.VMEM((CH,), jnp.int32),
        cb00=pltpu.VMEM((CH,), jnp.float32),
        cb01=pltpu.VMEM((CH,), jnp.float32),
        cb10=pltpu.VMEM((CH,), jnp.float32),
        cb11=pltpu.VMEM((CH,), jnp.float32),
        mb00=pltpu.VMEM((CH,), jnp.float32),
        mb01=pltpu.VMEM((CH,), jnp.float32),
        mb10=pltpu.VMEM((CH,), jnp.float32),
        mb11=pltpu.VMEM((CH,), jnp.float32),
        st0=pltpu.VMEM((16,), jnp.float32),
        st1=pltpu.VMEM((16,), jnp.float32),
        st2=pltpu.VMEM((16,), jnp.float32),
        st3=pltpu.VMEM((16,), jnp.float32),
        sem_in0=pltpu.SemaphoreType.DMA,
        sem_in1=pltpu.SemaphoreType.DMA,
        sem_out0=pltpu.SemaphoreType.DMA,
        sem_out1=pltpu.SemaphoreType.DMA,
    )

    def body(cT, pidx, aT, bT, dT, *rest):
        if write_m:
            mT, aggT, smT, sm2T = rest[:4]
            sc = rest[4:]
        else:
            aggT = rest[0]
            sc = rest[1:]
        (ta0, ta1, tb0, tb1, td0, td1, agg0, agg1, idx0, idx1,
         cb00, cb01, cb10, cb11, mb00, mb01, mb10, mb11,
         st0, st1, st2, st3,
         sem_in0, sem_in1, sem_out0, sem_out1) = sc
        sem_in = (sem_in0, sem_in1)
        sem_out = (sem_out0, sem_out1)
        idxb = (idx0, idx1)
        cbb = ((cb00, cb01), (cb10, cb11))
        mbb = ((mb00, mb01), (mb10, mb11))

        wid = lax.axis_index("s") * _NC + lax.axis_index("c")
        c0 = 2 * wid
        c1 = c0 + 1

        def row(base, length, k=None, ch=None):
            off = base * length
            if k is not None:
                off = off + k * ch
                length = ch
            return pl.ds(pl.multiple_of(off, 8), length)

        @pl.when(wid < n_tiles_active)
        def _active():
            pltpu.sync_copy(aT.at[row(c0, N)], ta0)
            pltpu.sync_copy(aT.at[row(c1, N)], ta1)
            pltpu.sync_copy(bT.at[row(c0, N)], tb0)
            pltpu.sync_copy(bT.at[row(c1, N)], tb1)
            pltpu.sync_copy(dT.at[row(c0, N)], td0)
            pltpu.sync_copy(dT.at[row(c1, N)], td1)

            zero16 = jnp.zeros((16,), jnp.float32)

            def zbody(i, _):
                sl = pl.ds(pl.multiple_of(i * 16, 16), 16)
                agg0[sl] = zero16
                agg1[sl] = zero16
                return 0
            lax.fori_loop(0, N // 16, zbody, 0)

            def issue_in(k, b):
                off = pl.ds(pl.multiple_of(k * CH, 8), CH)
                pltpu.async_copy(pidx.at[off], idxb[b], sem_in[b])
                pltpu.async_copy(cT.at[row(c0, E, k, CH)], cbb[b][0], sem_in[b])
                pltpu.async_copy(cT.at[row(c1, E, k, CH)], cbb[b][1], sem_in[b])

            def wait_in(k, b):
                off = pl.ds(pl.multiple_of(k * CH, 8), CH)
                pltpu.make_async_copy(pidx.at[off], idxb[b], sem_in[b]).wait()
                pltpu.make_async_copy(cT.at[row(c0, E, k, CH)], cbb[b][0], sem_in[b]).wait()
                pltpu.make_async_copy(cT.at[row(c1, E, k, CH)], cbb[b][1], sem_in[b]).wait()

            def issue_out(k, b):
                pltpu.async_copy(mbb[b][0], mT.at[row(c0, E, k, CH)], sem_out[b])
                pltpu.async_copy(mbb[b][1], mT.at[row(c1, E, k, CH)], sem_out[b])

            def wait_out(k, b):
                pltpu.make_async_copy(mbb[b][0], mT.at[row(c0, E, k, CH)], sem_out[b]).wait()
                pltpu.make_async_copy(mbb[b][1], mT.at[row(c1, E, k, CH)], sem_out[b]).wait()

            issue_in(0, 0)
            issue_in(1, 1)

            def chunk_work(b, acc):
                s0, q0, s1, q1 = acc

                def vbody(v, a):
                    s0, q0, s1, q1 = a
                    sl = pl.ds(pl.multiple_of(v * 16, 16), 16)
                    p = idxb[b][sl]
                    srcv = jnp.bitwise_and(p, jnp.int32(0xFFFF))
                    dstv = jnp.right_shift(p, jnp.int32(16))
                    m0 = (plsc.load_gather(ta0, [srcv])
                          + plsc.load_gather(tb0, [dstv]) + cbb[b][0][sl])
                    sg0 = 1.0 / (1.0 + jnp.exp(m0 * (-inv_scale)))
                    plsc.addupdate_scatter(
                        agg0, [srcv], sg0 * plsc.load_gather(td0, [dstv]))
                    m1 = (plsc.load_gather(ta1, [srcv])
                          + plsc.load_gather(tb1, [dstv]) + cbb[b][1][sl])
                    sg1 = 1.0 / (1.0 + jnp.exp(m1 * (-inv_scale)))
                    plsc.addupdate_scatter(
                        agg1, [srcv], sg1 * plsc.load_gather(td1, [dstv]))
                    if write_m:
                        mbb[b][0][sl] = m0
                        mbb[b][1][sl] = m1
                        s0 = s0 + m0
                        q0 = q0 + m0 * m0
                        s1 = s1 + m1
                        q1 = q1 + m1 * m1
                    return (s0, q0, s1, q1)

                return lax.fori_loop(0, NV, vbody, (s0, q0, s1, q1))

            def outer(kk, acc):
                for b in range(2):
                    k = 2 * kk + b
                    wait_in(k, b)
                    if write_m:
                        @pl.when(kk >= 1)
                        def _():
                            wait_out(k - 2, b)
                    acc = chunk_work(b, acc)
                    if write_m:
                        issue_out(k, b)

                    @pl.when(kk < NCH // 2 - 1)
                    def _():
                        issue_in(k + 2, b)
                return acc

            init = (jnp.zeros((16,), jnp.float32),) * 4
            s0, q0, s1, q1 = lax.fori_loop(0, NCH // 2, outer, init)

            if write_m:
                wait_out(NCH - 2, 0)
                wait_out(NCH - 1, 1)

            pltpu.sync_copy(agg0, aggT.at[row(c0, N)])
            pltpu.sync_copy(agg1, aggT.at[row(c1, N)])

            if write_m:
                st0[...] = s0
                st1[...] = s1
                st2[...] = q0
                st3[...] = q1
                pltpu.sync_copy(st0, smT.at[row(c0, 16)])
                pltpu.sync_copy(st1, smT.at[row(c1, 16)])
                pltpu.sync_copy(st2, sm2T.at[row(c0, 16)])
                pltpu.sync_copy(st3, sm2T.at[row(c1, 16)])

    return pl.kernel(
        body, out_type=out_type, mesh=mesh,
        scratch_types=list(scratch.values()),
        compiler_params=pltpu.CompilerParams(needs_layout_passes=False),
        name=f"sc_edge_pass_m{int(write_m)}")


def _dotT(w, xT):
    return lax.dot_general(w, xT, (((0,), (0,)), ((), ())),
                           preferred_element_type=jnp.float32)


def _node_pre(x, pre_N_W, pre_N_b, WgA, WgB, Wd, db, Ws, sb):
    N, DF = x.shape
    D = pre_N_W.shape[1]

    def body(x_ref, wn_ref, bn_ref, wa_ref, wb_ref, wd_ref, db_ref,
             ws_ref, sb_ref, ox_ref, a_ref, b_ref, d_ref, s_ref):
        xT = lax.dot_general(wn_ref[...], x_ref[...], (((0,), (1,)), ((), ())),
                             preferred_element_type=jnp.float32)
        ox = jnp.maximum(xT + bn_ref[...], 0.0)
        ox_ref[...] = ox
        a_ref[...] = _dotT(wa_ref[...], ox)
        b_ref[...] = _dotT(wb_ref[...], ox)
        d_ref[...] = _dotT(wd_ref[...], ox) + db_ref[...]
        s_ref[...] = _dotT(ws_ref[...], ox) + sb_ref[...]

    out = [jax.ShapeDtypeStruct((D, N), jnp.float32)] * 5
    return pl.pallas_call(body, out_shape=out)(
        x, pre_N_W, pre_N_b, WgA, WgB, Wd, db, Ws, sb)


def _edge_pre(edge_attr, pre_E_W, pre_E_b, WgE, gb, EB):
    E, DE = edge_attr.shape
    D = pre_E_W.shape[1]
    grid = (E // EB,)

    def body(ea_ref, we_ref, be_ref, wg_ref, gb_ref, oe_ref, c_ref):
        eT = lax.dot_general(we_ref[...], ea_ref[...], (((0,), (1,)), ((), ())),
                             preferred_element_type=jnp.float32)
        oe = jnp.maximum(eT + be_ref[...], 0.0)
        oe_ref[...] = oe
        c_ref[...] = _dotT(wg_ref[...], oe) + gb_ref[...]

    return pl.pallas_call(
        body,
        grid=grid,
        in_specs=[
            pl.BlockSpec((EB, DE), lambda i: (i, 0)),
            pl.BlockSpec((DE, D), lambda i: (0, 0)),
            pl.BlockSpec((D, 1), lambda i: (0, 0)),
            pl.BlockSpec((D, D), lambda i: (0, 0)),
            pl.BlockSpec((D, 1), lambda i: (0, 0)),
        ],
        out_specs=[
            pl.BlockSpec((D, EB), lambda i: (0, i)),
            pl.BlockSpec((D, EB), lambda i: (0, i)),
        ],
        out_shape=[jax.ShapeDtypeStruct((D, E), jnp.float32)] * 2,
    )(edge_attr, pre_E_W, pre_E_b, WgE, gb)


def _node_layer(S, aggT, prevx, cn_g, cn_b, bn_g, bn_b, nextw,
                sm=None, sm2=None, ce_g=None, ce_b=None, E=None):
    D, N = S.shape
    has_stats = sm is not None
    has_next = nextw is not None

    def body(*refs):
        it = iter(refs)
        s_ref = next(it)
        agg_ref = next(it)
        px_ref = next(it)
        cng_ref = next(it)
        cnb_ref = next(it)
        bng_ref = next(it)
        bnb_ref = next(it)
        if has_stats:
            sm_ref = next(it)
            sm2_ref = next(it)
            ceg_ref = next(it)
            ceb_ref = next(it)
        if has_next:
            wa_ref = next(it)
            wb_ref = next(it)
            wd_ref = next(it)
            db_ref = next(it)
            ws_ref = next(it)
            sb_ref = next(it)
        nx_ref = next(it)
        if has_stats:
            eaff_ref = next(it)
        if has_next:
            a_ref = next(it)
            b_ref = next(it)
            d_ref = next(it)
            so_ref = next(it)

        h = s_ref[...] + agg_ref[...]
        mu = jnp.mean(h, axis=1, keepdims=True)
        var = jnp.mean(h * h, axis=1, keepdims=True) - mu * mu
        z = (h - mu) * lax.rsqrt(var + EPS) * cng_ref[...] + cnb_ref[...]
        y = jnp.maximum(z, 0.0)
        mu2 = jnp.mean(y, axis=1, keepdims=True)
        var2 = jnp.mean(y * y, axis=1, keepdims=True) - mu2 * mu2
        ox = (y - mu2) * lax.rsqrt(var2 + EPS) * bng_ref[...] + bnb_ref[...]
        nx = ox + px_ref[...]
        nx_ref[...] = nx

        if has_stats:
            ssum = jnp.sum(sm_ref[...], axis=1, keepdims=True)
            qsum = jnp.sum(sm2_ref[...], axis=1, keepdims=True)
            mu1 = ssum / E
            var1 = qsum / E - mu1 * mu1
            s1 = ceg_ref[...] * lax.rsqrt(var1 + EPS)
            t1 = ceb_ref[...] - mu1 * s1
            eaff_ref[...] = jnp.concatenate([s1, t1], axis=1)

        if has_next:
            a_ref[...] = _dotT(wa_ref[...], nx)
            b_ref[...] = _dotT(wb_ref[...], nx)
            d_ref[...] = _dotT(wd_ref[...], nx) + db_ref[...]
            so_ref[...] = _dotT(ws_ref[...], nx) + sb_ref[...]

    args = [S, aggT, prevx, cn_g, cn_b, bn_g, bn_b]
    if has_stats:
        args += [sm, sm2, ce_g, ce_b]
    if has_next:
        args += list(nextw)
    out = [jax.ShapeDtypeStruct((D, N), jnp.float32)]
    if has_stats:
        out.append(jax.ShapeDtypeStruct((D, 2), jnp.float32))
    if has_next:
        out += [jax.ShapeDtypeStruct((D, N), jnp.float32)] * 4
    res = pl.pallas_call(body, out_shape=out)(*args)
    return res


def _edge_stats(mT, eaff, EB):
    D, E = mT.shape
    grid = (E // EB,)

    def body(m_ref, ea_ref, out_ref, acc_ref):
        i = pl.program_id(0)

        @pl.when(i == 0)
        def _():
            acc_ref[...] = jnp.zeros_like(acc_ref)

        s1 = ea_ref[:, 0:1]
        t1 = ea_ref[:, 1:2]
        y = jnp.maximum(m_ref[...] * s1 + t1, 0.0)
        sy = jnp.sum(y, axis=1, keepdims=True)
        sy2 = jnp.sum(y * y, axis=1, keepdims=True)
        acc_ref[...] += jnp.concatenate([sy, sy2], axis=1)

        @pl.when(i == pl.num_programs(0) - 1)
        def _():
            out_ref[...] = acc_ref[...]

    return pl.pallas_call(
        body,
        grid=grid,
        in_specs=[
            pl.BlockSpec((D, EB), lambda i: (0, i)),
            pl.BlockSpec((D, 2), lambda i: (0, 0)),
        ],
        out_specs=pl.BlockSpec((D, 2), lambda i: (0, 0)),
        out_shape=jax.ShapeDtypeStruct((D, 2), jnp.float32),
        scratch_shapes=[pltpu.VMEM((D, 2), jnp.float32)],
    )(mT, eaff)


def _edge_update(mT, oeT, eaff, sums2, be_g, be_b, WgE_next, gb_next, EB, E):
    D = mT.shape[0]
    E = int(E)
    grid = (E // EB,)
    fE = float(E)
    has_next = WgE_next is not None

    def body(*refs):
        if has_next:
            (m_ref, oe_ref, ea_ref, s2_ref, beg_ref, beb_ref, wg_ref, gb_ref,
             out_ref, c_ref) = refs
        else:
            m_ref, oe_ref, ea_ref, s2_ref, beg_ref, beb_ref, out_ref = refs
        s1 = ea_ref[:, 0:1]
        t1 = ea_ref[:, 1:2]
        y = jnp.maximum(m_ref[...] * s1 + t1, 0.0)
        mu2 = s2_ref[:, 0:1] / fE
        var2 = s2_ref[:, 1:2] / fE - mu2 * mu2
        s2 = beg_ref[...] * lax.rsqrt(var2 + EPS)
        t2 = beb_ref[...] - mu2 * s2
        oe = y * s2 + t2 + oe_ref[...]
        out_ref[...] = oe
        if has_next:
            c_ref[...] = _dotT(wg_ref[...], oe) + gb_ref[...]

    in_specs = [
        pl.BlockSpec((D, EB), lambda i: (0, i)),
        pl.BlockSpec((D, EB), lambda i: (0, i)),
        pl.BlockSpec((D, 2), lambda i: (0, 0)),
        pl.BlockSpec((D, 2), lambda i: (0, 0)),
        pl.BlockSpec((D, 1), lambda i: (0, 0)),
        pl.BlockSpec((D, 1), lambda i: (0, 0)),
    ]
    args = [mT, oeT, eaff, sums2, be_g, be_b]
    out_specs = [pl.BlockSpec((D, EB), lambda i: (0, i))]
    out_shape = [jax.ShapeDtypeStruct((D, E), jnp.float32)]
    if has_next:
        in_specs += [pl.BlockSpec((D, D), lambda i: (0, 0)),
                     pl.BlockSpec((D, 1), lambda i: (0, 0))]
        args += [WgE_next, gb_next]
        out_specs.append(pl.BlockSpec((D, EB), lambda i: (0, i)))
        out_shape.append(jax.ShapeDtypeStruct((D, E), jnp.float32))

    return pl.pallas_call(
        body, grid=grid, in_specs=in_specs, out_specs=out_specs,
        out_shape=out_shape)(*args)


def _final(out_x_T, batch2, post_W, post_b_row, out_W, out_b_11, NG):
    D, N = out_x_T.shape
    D2 = post_W.shape[1]

    def body(x_ref, b_ref, pw_ref, pb_ref, ow_ref, ob_ref, out_ref):
        gids = lax.broadcasted_iota(jnp.int32, (N, NG), 1)
        onehot = (b_ref[...] == gids).astype(jnp.float32)
        pooled = lax.dot_general(x_ref[...], onehot, (((1,), (0,)), ((), ())),
                                 preferred_element_type=jnp.float32)
        h1 = lax.dot_general(pooled, pw_ref[...], (((0,), (0,)), ((), ())),
                             preferred_element_type=jnp.float32)
        h1 = jnp.maximum(h1 + pb_ref[...], 0.0)
        out = lax.dot_general(h1, ow_ref[...], (((1,), (0,)), ((), ())),
                              preferred_element_type=jnp.float32)
        out_ref[...] = out + ob_ref[...]

    return pl.pallas_call(
        body, out_shape=jax.ShapeDtypeStruct((NG, 1), jnp.float32),
    )(out_x_T, batch2, post_W, post_b_row, out_W, out_b_11)


def kernel(x, edge_index, edge_attr, batch, pre_N_W, pre_N_b, pre_E_W, pre_E_b,
           gate_W, gate_b, src_W, src_b, dst_W, dst_b,
           cnbn_g, cnbn_b, cebn_g, cebn_b, bn_g, bn_b, be_g, be_b,
           post_W, post_b, out_W, out_b):
    N, DF = x.shape
    E, DE = edge_attr.shape
    D = pre_N_W.shape[1]
    GC = gate_W.shape[0]
    D2 = post_W.shape[1]
    NG = 64
    EB = next(eb for eb in range(12800, 127, -128) if E % eb == 0)
    inv_scale = 1.0 / math.sqrt(float(D))

    col = lambda v: v.reshape(-1, 1)

    packed = edge_index[0] | (edge_index[1] << 16)
    batch2 = batch.reshape(N, 1)

    WgA = [gate_W[i, :D] for i in range(GC)]
    WgB = [gate_W[i, D:2 * D] for i in range(GC)]
    WgE = [gate_W[i, 2 * D:] for i in range(GC)]
    gb = [col(gate_b[i]) for i in range(GC)]

    def nextw(i):
        return (WgA[i], WgB[i], dst_W[i], col(dst_b[i]), src_W[i],
                col(src_b[i]))

    out_x_T, A, B, Dg, S = _node_pre(
        x, pre_N_W, col(pre_N_b), *nextw(0))
    oeT, C = _edge_pre(edge_attr, pre_E_W, col(pre_E_b), WgE[0], gb[0], EB)

    sc_full = _sc_edge_pass(D, E, N, inv_scale, write_m=True)
    sc_last = _sc_edge_pass(D, E, N, inv_scale, write_m=False)

    for i in range(GC):
        last = i == GC - 1
        flat = lambda v: v.reshape(-1)
        if not last:
            mT, aggT, sm, sm2 = sc_full(flat(C), packed, flat(A), flat(B),
                                        flat(Dg))
            mT = mT.reshape(D, E)
            aggT = aggT.reshape(D, N)
            sm = sm.reshape(D, 16)
            sm2 = sm2.reshape(D, 16)
            nx, eaff, A, B, Dg, S = _node_layer(
                S, aggT, out_x_T, col(cnbn_g[i]), col(cnbn_b[i]),
                col(bn_g[i]), col(bn_b[i]), nextw(i + 1),
                sm=sm, sm2=sm2, ce_g=col(cebn_g[i]), ce_b=col(cebn_b[i]),
                E=float(E))
            sums2 = _edge_stats(mT, eaff, EB)
            oeT, C = _edge_update(mT, oeT, eaff, sums2, col(be_g[i]),
                                  col(be_b[i]), WgE[i + 1], gb[i + 1],
                                  EB, float(E))
        else:
            aggT = sc_last(flat(C), packed, flat(A), flat(B), flat(Dg))
            if isinstance(aggT, (list, tuple)):
                aggT = aggT[0]
            aggT = aggT.reshape(D, N)
            (nx,) = _node_layer(
                S, aggT, out_x_T, col(cnbn_g[i]), col(cnbn_b[i]),
                col(bn_g[i]), col(bn_b[i]), None)
        out_x_T = nx

    return _final(out_x_T, batch2, post_W, col(post_b).reshape(1, D2),
                  out_W, out_b.reshape(1, 1), NG)
